# Initial kernel scaffold; baseline (speedup 1.0000x reference)
import jax
import jax.numpy as jnp
from jax.experimental import pallas as pl


def kernel(x, W_pre, b_pre, Wc1, bc1, Wc2, bc2, Wc3, bc3, Wc4, bc4, W_post, b_post, edge_index, batch):
    raise NotImplementedError("write your pallas kernel here")


# speedup vs baseline: 16.2294x; 16.2294x over previous
import functools

import jax
import jax.numpy as jnp
from jax import lax
from jax.experimental import pallas as pl
from jax.experimental.pallas import tpu as pltpu
from jax.experimental.pallas import tpu_sc as plsc

_N = 10000
_E = 320000
_D = 128
_C = 10
_B = 64

_NC = 2
_NS = 16
_NW = _NC * _NS
_EPW = _E // _NW
_K = 125
_NCH = _EPW // _K
_RPT = _N // _NS

_BLK = 1000
_G = _N // _BLK

_OWN = 624
_TAIL = _N - _OWN * _NS


def _sc_mesh():
  return plsc.VectorSubcoreMesh(
      core_axis_name="c", subcore_axis_name="s",
      num_cores=_NC, num_subcores=_NS)


def _sc_deg_body(dst_hbm, zeros_hbm, ones_hbm, out_hbm, acc, dst_v, ones_v,
                 sem):
  cid = lax.axis_index("c")
  sid = lax.axis_index("s")
  wid = cid * _NS + sid

  pltpu.sync_copy(zeros_hbm.at[pl.ds(0, _OWN)], acc.at[pl.ds(sid * _OWN, _OWN)])

  @pl.when(sid < 2)
  def _():
    pltpu.sync_copy(zeros_hbm.at[pl.ds(0, 8)],
                    acc.at[pl.ds(_OWN * _NS + sid * 8, 8)])

  pltpu.sync_copy(ones_hbm, ones_v)
  pltpu.sync_copy(dst_hbm.at[wid], dst_v)
  plsc.subcore_barrier()

  def step(j, carry):
    pltpu.sync_copy(ones_v, acc.at[dst_v.at[j]], add=True)
    return carry

  lax.fori_loop(0, _NCH, step, 0)
  plsc.subcore_barrier()
  pltpu.sync_copy(acc.at[pl.ds(sid * _OWN, _OWN)],
                  out_hbm.at[cid, pl.ds(sid * _OWN, _OWN)])

  @pl.when(sid < 2)
  def _():
    pltpu.sync_copy(acc.at[pl.ds(_OWN * _NS + sid * 8, 8)],
                    out_hbm.at[cid, pl.ds(_OWN * _NS + sid * 8, 8)])


def _sc_degree(dst3, zeros, ones):
  return pl.kernel(
      _sc_deg_body,
      out_type=jax.ShapeDtypeStruct((_NC, _N, _D), jnp.float32),
      mesh=_sc_mesh(),
      scratch_types=[
          pltpu.VMEM_SHARED((_N, _D), jnp.float32),
          pltpu.VMEM((_NCH, _K), jnp.int32),
          pltpu.VMEM((_K, _D), jnp.float32),
          pltpu.SemaphoreType.DMA,
      ],
  )(dst3, zeros, ones)


def _sc_agg_body(u_hbm, src_hbm, dst_hbm, zeros_hbm, out_hbm, acc, src_v,
                 dst_v, rows_v, sem):
  cid = lax.axis_index("c")
  sid = lax.axis_index("s")
  wid = cid * _NS + sid

  pltpu.sync_copy(zeros_hbm.at[pl.ds(0, _OWN)], acc.at[pl.ds(sid * _OWN, _OWN)])

  @pl.when(sid < 2)
  def _():
    pltpu.sync_copy(zeros_hbm.at[pl.ds(0, 8)],
                    acc.at[pl.ds(_OWN * _NS + sid * 8, 8)])

  pltpu.sync_copy(src_hbm.at[wid], src_v)
  pltpu.sync_copy(dst_hbm.at[wid], dst_v)
  plsc.subcore_barrier()

  def step(j, carry):
    pltpu.async_copy(u_hbm.at[src_v.at[j]], rows_v, sem).wait()
    pltpu.sync_copy(rows_v, acc.at[dst_v.at[j]], add=True)
    return carry

  lax.fori_loop(0, _NCH, step, 0)
  plsc.subcore_barrier()
  pltpu.sync_copy(acc.at[pl.ds(sid * _OWN, _OWN)],
                  out_hbm.at[cid, pl.ds(sid * _OWN, _OWN)])

  @pl.when(sid < 2)
  def _():
    pltpu.sync_copy(acc.at[pl.ds(_OWN * _NS + sid * 8, 8)],
                    out_hbm.at[cid, pl.ds(_OWN * _NS + sid * 8, 8)])


def _sc_aggregate(u, src3, dst3, zeros):
  return pl.kernel(
      _sc_agg_body,
      out_type=jax.ShapeDtypeStruct((_NC, _N, _D), jnp.float32),
      mesh=_sc_mesh(),
      scratch_types=[
          pltpu.VMEM_SHARED((_N, _D), jnp.float32),
          pltpu.VMEM((_NCH, _K), jnp.int32),
          pltpu.VMEM((_NCH, _K), jnp.int32),
          pltpu.VMEM((_K, _D), jnp.float32),
          pltpu.SemaphoreType.DMA,
      ],
  )(u, src3, dst3, zeros)


def _dinv_of(deg_ref):
  deg = deg_ref[0, :, 0:1] + deg_ref[1, :, 0:1] + 1.0
  return lax.rsqrt(deg)


def _t0_body(deg_ref, x_ref, wpre_ref, bpre_ref, wc1_ref, u_ref):
  dinv = _dinv_of(deg_ref)
  h0 = jnp.dot(x_ref[...], wpre_ref[...],
               preferred_element_type=jnp.float32) + bpre_ref[...]
  u_ref[...] = jnp.dot(h0, wc1_ref[...],
                       preferred_element_type=jnp.float32) * dinv


def _tmid_body(deg_ref, agg_ref, u_ref, b_ref, w_ref, out_ref):
  dinv = _dinv_of(deg_ref)
  h = jnp.maximum(
      (agg_ref[0] + agg_ref[1] + u_ref[...]) * dinv + b_ref[...], 0.0)
  out_ref[...] = jnp.dot(h, w_ref[...],
                         preferred_element_type=jnp.float32) * dinv


def _pool_body(deg_ref, agg_ref, u_ref, b_ref, batch_ref, wpost_ref,
               bpost_ref, y_ref, ssum, cnt):
  i = pl.program_id(0)
  dinv = _dinv_of(deg_ref)
  h = jnp.maximum(
      (agg_ref[0] + agg_ref[1] + u_ref[...]) * dinv + b_ref[...], 0.0)
  oh = (lax.broadcasted_iota(jnp.int32, (_BLK, _B), 1)
        == batch_ref[...]).astype(jnp.float32)
  dn = (((0,), (0,)), ((), ()))
  ps = lax.dot_general(oh, h, dn, preferred_element_type=jnp.float32)
  pc = lax.dot_general(oh, jnp.ones_like(h), dn,
                       preferred_element_type=jnp.float32)

  @pl.when(i == 0)
  def _():
    ssum[...] = ps
    cnt[...] = pc

  @pl.when(i > 0)
  def _():
    ssum[...] += ps
    cnt[...] += pc

  @pl.when(i == _G - 1)
  def _():
    g = ssum[...] / jnp.maximum(cnt[...], 1.0)
    logits = jnp.dot(g, wpost_ref[...],
                     preferred_element_type=jnp.float32) + bpost_ref[...]
    m = jnp.max(logits, axis=1, keepdims=True)
    lse = jnp.log(jnp.sum(jnp.exp(logits - m), axis=1, keepdims=True))
    y_ref[...] = logits - m - lse


_deg_spec = pl.BlockSpec((_NC, _BLK, _D), lambda i: (0, i, 0))
_agg_spec = pl.BlockSpec((_NC, _BLK, _D), lambda i: (0, i, 0))
_row_spec = pl.BlockSpec((_BLK, _D), lambda i: (i, 0))


def _full(shape):
  return pl.BlockSpec(shape, lambda i: tuple(0 for _ in shape))


def _t0(degp, x, w_pre, b_pre, wc1):
  return pl.pallas_call(
      _t0_body,
      grid=(_G,),
      in_specs=[_deg_spec, _row_spec, _full((_D, _D)), _full((1, _D)),
                _full((_D, _D))],
      out_specs=_row_spec,
      out_shape=jax.ShapeDtypeStruct((_N, _D), jnp.float32),
  )(degp, x, w_pre, b_pre, wc1)


def _tmid(degp, aggp, u, b, w):
  return pl.pallas_call(
      _tmid_body,
      grid=(_G,),
      in_specs=[_deg_spec, _agg_spec, _row_spec, _full((1, _D)),
                _full((_D, _D))],
      out_specs=_row_spec,
      out_shape=jax.ShapeDtypeStruct((_N, _D), jnp.float32),
  )(degp, aggp, u, b, w)


def _pool(degp, aggp, u, b, batch2, w_post, b_post):
  return pl.pallas_call(
      _pool_body,
      grid=(_G,),
      in_specs=[_deg_spec, _agg_spec, _row_spec, _full((1, _D)),
                pl.BlockSpec((_BLK, 1), lambda i: (i, 0)),
                _full((_D, _C)), _full((1, _C))],
      out_specs=_full((_B, _C)),
      out_shape=jax.ShapeDtypeStruct((_B, _C), jnp.float32),
      scratch_shapes=[pltpu.VMEM((_B, _D), jnp.float32),
                      pltpu.VMEM((_B, _D), jnp.float32)],
  )(degp, aggp, u, b, batch2, w_post, b_post)


def kernel(x, W_pre, b_pre, Wc1, bc1, Wc2, bc2, Wc3, bc3, Wc4, bc4, W_post,
           b_post, edge_index, batch):
  src3 = edge_index[0].reshape(_NW, _NCH, _K)
  dst3 = edge_index[1].reshape(_NW, _NCH, _K)
  batch2 = batch.reshape(_N, 1)
  b_pre2 = b_pre.reshape(1, _D)
  bc12 = bc1.reshape(1, _D)
  bc22 = bc2.reshape(1, _D)
  bc32 = bc3.reshape(1, _D)
  bc42 = bc4.reshape(1, _D)
  b_post2 = b_post.reshape(1, _C)
  zeros = jnp.zeros((_OWN, _D), jnp.float32)
  ones = jnp.ones((_K, _D), jnp.float32)

  degp = _sc_degree(dst3, zeros, ones)

  u1 = _t0(degp, x, W_pre, b_pre2, Wc1)
  a1 = _sc_aggregate(u1, src3, dst3, zeros)
  u2 = _tmid(degp, a1, u1, bc12, Wc2)
  a2 = _sc_aggregate(u2, src3, dst3, zeros)
  u3 = _tmid(degp, a2, u2, bc22, Wc3)
  a3 = _sc_aggregate(u3, src3, dst3, zeros)
  u4 = _tmid(degp, a3, u3, bc32, Wc4)
  a4 = _sc_aggregate(u4, src3, dst3, zeros)
  return _pool(degp, a4, u4, bc42, batch2, W_post, b_post2)
